# baseline (speedup 1.0000x reference)
import functools
import jax
import jax.numpy as jnp
from jax import lax
from jax.experimental import pallas as pl
from jax.experimental.pallas import tpu as pltpu
from jax.experimental.pallas import tpu_sc as plsc

H, W = 180, 240
KH, KW = 3, 3
P = H * W
D = 128
NE = 32768
H_OUT, W_OUT = H - KH + 1, W - KW + 1
NRF = H_OUT * W_OUT

NTILES = 16
EPT = NE // NTILES
NCHUNK = 4
CSZ = P // NCHUNK
CPAD = 10880
ZPT = CPAD // NTILES
WPT = 672
WTAIL = CSZ - NTILES * WPT
BATCH = 128
NBATCH = EPT // BATCH

RPS = 2
NQ = 2
NSTEP = H_OUT // RPS


def _spans(total, step):
    out, off = [], 0
    while off < total:
        n = min(step, total - off)
        out.append((off, n))
        off += n
    return out


def _seg_sum_body(feats, pix, zeros, out, acc, pixv, idx2d, rows, zbuf):
    c = lax.axis_index("c")
    s = lax.axis_index("s")
    base_e = s * EPT
    pltpu.sync_copy(pix.at[pl.ds(base_e, EPT)], pixv)
    pltpu.sync_copy(zeros, zbuf)
    for k in range(NCHUNK // 2):
        start = (2 * c + k) * CSZ
        zb = s * ZPT
        for off, n in _spans(ZPT, BATCH):
            pltpu.sync_copy(zbuf.at[pl.ds(0, n)], acc.at[pl.ds(zb + off, n)])
        plsc.subcore_barrier()
        for b in range(NBATCH):
            for j in range(BATCH // 16):
                v = pixv[pl.ds(b * BATCH + j * 16, 16)]
                rel = v - start
                ok = (rel >= 0) & (rel < CSZ)
                idx2d[b, pl.ds(j * 16, 16)] = jnp.where(ok, rel, CSZ)
            pltpu.sync_copy(feats.at[pl.ds(base_e + b * BATCH, BATCH)], rows)
            pltpu.sync_copy(rows, acc.at[idx2d.at[b]], add=True)
        plsc.subcore_barrier()
        ob = s * WPT
        for off, n in _spans(WPT, BATCH):
            pltpu.sync_copy(acc.at[pl.ds(ob + off, n)], rows.at[pl.ds(0, n)])
            pltpu.sync_copy(rows.at[pl.ds(0, n)],
                            out.at[pl.ds(start + ob + off, n)])
        @pl.when(s == 0)
        def _tail():
            tb = NTILES * WPT
            pltpu.sync_copy(acc.at[pl.ds(tb, WTAIL)], rows.at[pl.ds(0, WTAIL)])
            pltpu.sync_copy(rows.at[pl.ds(0, WTAIL)],
                            out.at[pl.ds(start + tb, WTAIL)])
        plsc.subcore_barrier()


_seg_sum = functools.partial(
    pl.kernel,
    out_type=jax.ShapeDtypeStruct((P, D), jnp.float32),
    mesh=plsc.VectorSubcoreMesh(core_axis_name="c", subcore_axis_name="s"),
    scratch_types=[
        pltpu.VMEM_SHARED((CPAD, D), jnp.float32),
        pltpu.VMEM((EPT,), jnp.int32),
        pltpu.VMEM((NBATCH, BATCH), jnp.int32),
        pltpu.VMEM((BATCH, D), jnp.float32),
        pltpu.VMEM((BATCH, D), jnp.float32),
    ],
)(_seg_sum_body)


def _im2col_body(pf_ref, out_ref, buf, sem0, sem1):
    i = pl.program_id(0)
    slot = lax.rem(i, NQ)
    sems = (sem0, sem1)

    def copy(step, q):
        return pltpu.make_async_copy(
            buf.at[q],
            out_ref.at[pl.ds(step * RPS * W_OUT, RPS * W_OUT)],
            sems[q])

    for q in range(NQ):
        @pl.when((i >= NQ) & (slot == q))
        def _wait(q=q):
            copy(i - NQ, q).wait()

        @pl.when(slot == q)
        def _compute_and_send(q=q):
            for rr in range(RPS):
                for ky in range(KH):
                    r = pf_ref[RPS * i + rr + ky]
                    for kx in range(KW):
                        buf[q, rr * W_OUT:(rr + 1) * W_OUT,
                            ky * KW + kx] = r[kx:kx + W_OUT, :]
            copy(i, q).start()

    @pl.when(i == NSTEP - 1)
    def _drain():
        for step in range(NSTEP - NQ, NSTEP):
            copy(step, step % NQ).wait()


def kernel(event_feats, event_pixels):
    pix = event_pixels.astype(jnp.int32)
    zeros = jnp.zeros((BATCH, D), jnp.float32)
    pixel_feats = _seg_sum(event_feats, pix, zeros)
    rf = pl.pallas_call(
        _im2col_body,
        grid=(NSTEP,),
        in_specs=[pl.BlockSpec((H, W, D), lambda i: (0, 0, 0))],
        out_specs=pl.BlockSpec(memory_space=pltpu.MemorySpace.HBM),
        out_shape=jax.ShapeDtypeStruct((NRF, KH * KW, D), jnp.float32),
        scratch_shapes=[
            pltpu.VMEM((NQ, RPS * W_OUT, KH * KW, D), jnp.float32),
            pltpu.SemaphoreType.DMA,
            pltpu.SemaphoreType.DMA,
        ],
    )(pixel_feats.reshape(H, W, D))
    return rf


# speedup vs baseline: 1.2437x; 1.2437x over previous
import functools
import jax
import jax.numpy as jnp
from jax import lax
from jax.experimental import pallas as pl
from jax.experimental.pallas import tpu as pltpu
from jax.experimental.pallas import tpu_sc as plsc

H, W = 180, 240
KH, KW = 3, 3
P = H * W
D = 128
NE = 32768
H_OUT, W_OUT = H - KH + 1, W - KW + 1
NRF = H_OUT * W_OUT

NTILES = 16
EPT = NE // NTILES
NCHUNK = 4
CSZ = P // NCHUNK
CPAD = 10880
ZPT = CPAD // NTILES
WPT = 672
WTAIL = CSZ - NTILES * WPT
BATCH = 128
NBATCH = EPT // BATCH

NTAP = KH * KW


def _spans(total, step):
    out, off = [], 0
    while off < total:
        n = min(step, total - off)
        out.append((off, n))
        off += n
    return out


def _seg_sum_body(feats, pix, zeros, out, acc, pixv, idx2d, rows, zbuf):
    c = lax.axis_index("c")
    s = lax.axis_index("s")
    base_e = s * EPT
    pltpu.sync_copy(pix.at[pl.ds(base_e, EPT)], pixv)
    pltpu.sync_copy(zeros, zbuf)
    for k in range(NCHUNK // 2):
        start = (2 * c + k) * CSZ
        zb = s * ZPT
        for off, n in _spans(ZPT, BATCH):
            pltpu.sync_copy(zbuf.at[pl.ds(0, n)], acc.at[pl.ds(zb + off, n)])
        plsc.subcore_barrier()
        for b in range(NBATCH):
            for j in range(BATCH // 16):
                v = pixv[pl.ds(b * BATCH + j * 16, 16)]
                rel = v - start
                ok = (rel >= 0) & (rel < CSZ)
                idx2d[b, pl.ds(j * 16, 16)] = jnp.where(ok, rel, CSZ)
            pltpu.sync_copy(feats.at[pl.ds(base_e + b * BATCH, BATCH)], rows)
            pltpu.sync_copy(rows, acc.at[idx2d.at[b]], add=True)
        plsc.subcore_barrier()
        ob = s * WPT
        for off, n in _spans(WPT, BATCH):
            pltpu.sync_copy(acc.at[pl.ds(ob + off, n)], rows.at[pl.ds(0, n)])
            pltpu.sync_copy(rows.at[pl.ds(0, n)],
                            out.at[pl.ds(start + ob + off, n)])
        @pl.when(s == 0)
        def _tail():
            tb = NTILES * WPT
            pltpu.sync_copy(acc.at[pl.ds(tb, WTAIL)], rows.at[pl.ds(0, WTAIL)])
            pltpu.sync_copy(rows.at[pl.ds(0, WTAIL)],
                            out.at[pl.ds(start + tb, WTAIL)])
        plsc.subcore_barrier()


_seg_sum = functools.partial(
    pl.kernel,
    out_type=jax.ShapeDtypeStruct((P, D), jnp.float32),
    mesh=plsc.VectorSubcoreMesh(core_axis_name="c", subcore_axis_name="s"),
    scratch_types=[
        pltpu.VMEM_SHARED((CPAD, D), jnp.float32),
        pltpu.VMEM((EPT,), jnp.int32),
        pltpu.VMEM((NBATCH, BATCH), jnp.int32),
        pltpu.VMEM((BATCH, D), jnp.float32),
        pltpu.VMEM((BATCH, D), jnp.float32),
    ],
)(_seg_sum_body)


def _im2col_body(pf_ref, out_ref, sems):
    def copy(ky, kx):
        tap = ky * KW + kx
        return pltpu.make_async_copy(
            pf_ref.at[pl.ds(ky, H_OUT), pl.ds(kx, W_OUT), :],
            out_ref.at[:, :, tap, :],
            sems.at[tap])

    for ky in range(KH):
        for kx in range(KW):
            copy(ky, kx).start()
    for ky in range(KH):
        for kx in range(KW):
            copy(ky, kx).wait()


def kernel(event_feats, event_pixels):
    pix = event_pixels.astype(jnp.int32)
    zeros = jnp.zeros((BATCH, D), jnp.float32)
    pixel_feats = _seg_sum(event_feats, pix, zeros)
    rf = pl.pallas_call(
        _im2col_body,
        grid=(1,),
        in_specs=[pl.BlockSpec((H, W, D), lambda i: (0, 0, 0))],
        out_specs=pl.BlockSpec(memory_space=pltpu.MemorySpace.HBM),
        out_shape=jax.ShapeDtypeStruct((H_OUT, W_OUT, KH * KW, D),
                                       jnp.float32),
        scratch_shapes=[pltpu.SemaphoreType.DMA((NTAP,))],
    )(pixel_feats.reshape(H, W, D))
    return rf.reshape(NRF, KH * KW, D)
